# baseline (device time: 61602 ns/iter reference)
import jax
import jax.numpy as jnp
from jax import lax
from jax.experimental import pallas as pl
from jax.experimental.pallas import tpu as pltpu

NS = 32
LAG = 6
NC = 2


def kernel(x):
    _, m, n2 = x.shape
    n_half = n2 // 2
    mh = m // 2
    rows = mh // NS
    crows = mh // NC
    spc = NS // NC

    def body(x_ref, out_ref, ld_buf, big_buf, x_send, x_recv, y_recv,
             ld_sems, big_sems, xs_send, xs_recv, ys_send, ys_recv):
        my_x = lax.axis_index("x")
        my_y = lax.axis_index("y")
        my_z = lax.axis_index("z")
        x_nbr = (1 - my_x, my_y, my_z)
        y_nbr = (my_x, 1 - my_y, my_z)

        p1_off = my_y * mh
        p2_off = (1 - my_y) * mh

        barrier = pltpu.get_barrier_semaphore()
        for nbr in (x_nbr, y_nbr):
            pl.semaphore_signal(
                barrier, inc=1, device_id=nbr,
                device_id_type=pl.DeviceIdType.MESH,
            )
        pl.semaphore_wait(barrier, 2)

        def rdma_x(s):
            r = pl.ds(s * rows, rows)
            return pltpu.make_async_remote_copy(
                src_ref=x_send.at[r, :],
                dst_ref=x_recv.at[r, :],
                send_sem=xs_send.at[s],
                recv_sem=xs_recv.at[s],
                device_id=x_nbr,
                device_id_type=pl.DeviceIdType.MESH,
            )

        def rdma_y(s):
            r = pl.ds(s * rows, rows)
            return pltpu.make_async_remote_copy(
                src_ref=x_recv.at[r, :],
                dst_ref=y_recv.at[r, :],
                send_sem=ys_send.at[s],
                recv_sem=ys_recv.at[s],
                device_id=y_nbr,
                device_id_type=pl.DeviceIdType.MESH,
            )

        def go(my_col, their_col):
            def start_load(s, slot):
                return pltpu.make_async_copy(
                    x_ref.at[0, pl.ds(p1_off + s * rows, rows), their_col],
                    ld_buf.at[slot],
                    ld_sems.at[slot],
                )

            start_load(0, 0).start()
            for s in range(NS):
                if s + 1 < NS:
                    start_load(s + 1, (s + 1) % 2).start()
                start_load(s, s % 2).wait()
                x_send[pl.ds(s * rows, rows), :] = (
                    ld_buf[s % 2].astype(jnp.bfloat16))
                rdma_x(s).start()
                b = s - LAG
                if b >= 0:
                    rdma_x(b).wait_recv()
                    rdma_y(b).start()
            for b in range(NS - LAG, NS):
                rdma_x(b).wait_recv()
                rdma_y(b).start()

            def big_load(c, slot):
                row0 = jnp.where(c < NC, p1_off + c * crows,
                                 p2_off + (c - NC) * crows)
                return pltpu.make_async_copy(
                    x_ref.at[0, pl.ds(row0, crows), my_col],
                    big_buf.at[slot],
                    big_sems.at[slot],
                )

            big_load(0, 0).start()
            for c in range(2 * NC):
                if c + 1 < 2 * NC:
                    big_load(c + 1, (c + 1) % 2).start()
                big_load(c, c % 2).wait()
                rc = pl.ds((c % NC) * crows, crows)
                if c < NC:
                    out_ref[pl.ds(p1_off + c * crows, crows), :] = (
                        big_buf[c % 2].astype(jnp.bfloat16) + x_recv[rc, :])
                else:
                    for s in range((c - NC) * spc, (c - NC + 1) * spc):
                        rdma_y(s).wait_recv()
                    out_ref[pl.ds(p2_off + (c - NC) * crows, crows), :] = (
                        big_buf[c % 2].astype(jnp.bfloat16) + y_recv[rc, :])

        lo = slice(0, n_half)
        hi = slice(n_half, n2)

        @pl.when(my_x == 0)
        def _():
            go(lo, hi)

        @pl.when(my_x == 1)
        def _():
            go(hi, lo)

        for s in range(NS):
            rdma_x(s).wait_send()
            rdma_y(s).wait_send()

    return pl.pallas_call(
        body,
        out_shape=jax.ShapeDtypeStruct((m, n_half), jnp.bfloat16),
        in_specs=[pl.BlockSpec(memory_space=pl.ANY)],
        out_specs=pl.BlockSpec(memory_space=pltpu.VMEM),
        scratch_shapes=[
            pltpu.VMEM((2, rows, n_half), jnp.float32),
            pltpu.VMEM((2, mh // NC, n_half), jnp.float32),
            pltpu.VMEM((mh, n_half), jnp.bfloat16),
            pltpu.VMEM((mh, n_half), jnp.bfloat16),
            pltpu.VMEM((mh, n_half), jnp.bfloat16),
            pltpu.SemaphoreType.DMA((2,)),
            pltpu.SemaphoreType.DMA((2,)),
            pltpu.SemaphoreType.DMA((NS,)),
            pltpu.SemaphoreType.DMA((NS,)),
            pltpu.SemaphoreType.DMA((NS,)),
            pltpu.SemaphoreType.DMA((NS,)),
        ],
        compiler_params=pltpu.CompilerParams(collective_id=0),
    )(x)


# device time: 61198 ns/iter; 1.0066x vs baseline; 1.0066x over previous
import jax
import jax.numpy as jnp
from jax import lax
from jax.experimental import pallas as pl
from jax.experimental.pallas import tpu as pltpu

NS = 16
LAG = 4
NC = 2


def kernel(x):
    _, m, n2 = x.shape
    n_half = n2 // 2
    mh = m // 2
    rows = mh // NS
    crows = mh // NC
    spc = NS // NC

    def body(x_ref, out_ref, ld_buf, big_buf, x_send, x_recv, y_recv,
             ld_sems, big_sems, xs_send, xs_recv, ys_send, ys_recv):
        my_x = lax.axis_index("x")
        my_y = lax.axis_index("y")
        my_z = lax.axis_index("z")
        x_nbr = (1 - my_x, my_y, my_z)
        y_nbr = (my_x, 1 - my_y, my_z)

        p1_off = my_y * mh
        p2_off = (1 - my_y) * mh

        barrier = pltpu.get_barrier_semaphore()
        for nbr in (x_nbr, y_nbr):
            pl.semaphore_signal(
                barrier, inc=1, device_id=nbr,
                device_id_type=pl.DeviceIdType.MESH,
            )
        pl.semaphore_wait(barrier, 2)

        def rdma_x(s):
            r = pl.ds(s * rows, rows)
            return pltpu.make_async_remote_copy(
                src_ref=x_send.at[r, :],
                dst_ref=x_recv.at[r, :],
                send_sem=xs_send.at[s],
                recv_sem=xs_recv.at[s],
                device_id=x_nbr,
                device_id_type=pl.DeviceIdType.MESH,
            )

        def rdma_y(s):
            r = pl.ds(s * rows, rows)
            return pltpu.make_async_remote_copy(
                src_ref=x_recv.at[r, :],
                dst_ref=y_recv.at[r, :],
                send_sem=ys_send.at[s],
                recv_sem=ys_recv.at[s],
                device_id=y_nbr,
                device_id_type=pl.DeviceIdType.MESH,
            )

        def go(my_col, their_col):
            def start_load(s, slot):
                return pltpu.make_async_copy(
                    x_ref.at[0, pl.ds(p1_off + s * rows, rows), their_col],
                    ld_buf.at[slot],
                    ld_sems.at[slot],
                )

            start_load(0, 0).start()
            for s in range(NS):
                if s + 1 < NS:
                    start_load(s + 1, (s + 1) % 2).start()
                start_load(s, s % 2).wait()
                x_send[pl.ds(s * rows, rows), :] = (
                    ld_buf[s % 2].astype(jnp.bfloat16))
                rdma_x(s).start()
                b = s - LAG
                if b >= 0:
                    rdma_x(b).wait_recv()
                    rdma_y(b).start()
            for b in range(NS - LAG, NS):
                rdma_x(b).wait_recv()
                rdma_y(b).start()

            def big_load(c, slot):
                row0 = jnp.where(c < NC, p1_off + c * crows,
                                 p2_off + (c - NC) * crows)
                return pltpu.make_async_copy(
                    x_ref.at[0, pl.ds(row0, crows), my_col],
                    big_buf.at[slot],
                    big_sems.at[slot],
                )

            big_load(0, 0).start()
            for c in range(2 * NC):
                if c + 1 < 2 * NC:
                    big_load(c + 1, (c + 1) % 2).start()
                big_load(c, c % 2).wait()
                rc = pl.ds((c % NC) * crows, crows)
                if c < NC:
                    out_ref[pl.ds(p1_off + c * crows, crows), :] = (
                        big_buf[c % 2].astype(jnp.bfloat16) + x_recv[rc, :])
                else:
                    for i, s in enumerate(
                            range((c - NC) * spc, (c - NC + 1) * spc)):
                        rdma_y(s).wait_recv()
                        rs = pl.ds(s * rows, rows)
                        out_ref[pl.ds(p2_off + (c - NC) * crows
                                      + i * rows, rows), :] = (
                            big_buf[c % 2, i * rows:(i + 1) * rows, :]
                            .astype(jnp.bfloat16) + y_recv[rs, :])

        lo = slice(0, n_half)
        hi = slice(n_half, n2)

        @pl.when(my_x == 0)
        def _():
            go(lo, hi)

        @pl.when(my_x == 1)
        def _():
            go(hi, lo)

        for s in range(NS):
            rdma_x(s).wait_send()
            rdma_y(s).wait_send()

    return pl.pallas_call(
        body,
        out_shape=jax.ShapeDtypeStruct((m, n_half), jnp.bfloat16),
        in_specs=[pl.BlockSpec(memory_space=pl.ANY)],
        out_specs=pl.BlockSpec(memory_space=pltpu.VMEM),
        scratch_shapes=[
            pltpu.VMEM((2, rows, n_half), jnp.float32),
            pltpu.VMEM((2, mh // NC, n_half), jnp.float32),
            pltpu.VMEM((mh, n_half), jnp.bfloat16),
            pltpu.VMEM((mh, n_half), jnp.bfloat16),
            pltpu.VMEM((mh, n_half), jnp.bfloat16),
            pltpu.SemaphoreType.DMA((2,)),
            pltpu.SemaphoreType.DMA((2,)),
            pltpu.SemaphoreType.DMA((NS,)),
            pltpu.SemaphoreType.DMA((NS,)),
            pltpu.SemaphoreType.DMA((NS,)),
            pltpu.SemaphoreType.DMA((NS,)),
        ],
        compiler_params=pltpu.CompilerParams(collective_id=0),
    )(x)


# device time: 58221 ns/iter; 1.0581x vs baseline; 1.0511x over previous
import jax
import jax.numpy as jnp
from jax import lax
from jax.experimental import pallas as pl
from jax.experimental.pallas import tpu as pltpu

NS = 16
GS = 4
NG = NS // GS


def kernel(x):
    _, m, n2 = x.shape
    n_half = n2 // 2
    mh = m // 2
    rows = mh // NS
    grows = mh // NG

    def body(x_ref, out_ref, ld_big, big_buf, x_send, x_recv, y_recv,
             ldg_sems, big_sems, xs_send, xs_recv, ys_send, ys_recv):
        my_x = lax.axis_index("x")
        my_y = lax.axis_index("y")
        my_z = lax.axis_index("z")
        x_nbr = (1 - my_x, my_y, my_z)
        y_nbr = (my_x, 1 - my_y, my_z)

        p1_off = my_y * mh
        p2_off = (1 - my_y) * mh

        barrier = pltpu.get_barrier_semaphore()
        for nbr in (x_nbr, y_nbr):
            pl.semaphore_signal(
                barrier, inc=1, device_id=nbr,
                device_id_type=pl.DeviceIdType.MESH,
            )
        pl.semaphore_wait(barrier, 2)

        def rdma_x(s):
            r = pl.ds(s * rows, rows)
            return pltpu.make_async_remote_copy(
                src_ref=x_send.at[r, :],
                dst_ref=x_recv.at[r, :],
                send_sem=xs_send.at[s],
                recv_sem=xs_recv.at[s],
                device_id=x_nbr,
                device_id_type=pl.DeviceIdType.MESH,
            )

        def rdma_y(s):
            r = pl.ds(s * rows, rows)
            return pltpu.make_async_remote_copy(
                src_ref=x_recv.at[r, :],
                dst_ref=y_recv.at[r, :],
                send_sem=ys_send.at[s],
                recv_sem=ys_recv.at[s],
                device_id=y_nbr,
                device_id_type=pl.DeviceIdType.MESH,
            )

        def go(my_col, their_col):
            def stage_ld(g, slot):
                return pltpu.make_async_copy(
                    x_ref.at[0, pl.ds(p1_off + g * grows, grows), their_col],
                    ld_big.at[slot],
                    ldg_sems.at[slot],
                )

            seq = [("P", 0), ("P", 1), ("Q", 0), ("P", 2), ("Q", 1),
                   ("P", 3), ("Q", 2), ("Q", 3)]

            def add_ld(i):
                kind, c = seq[i]
                off = p1_off if kind == "P" else p2_off
                return pltpu.make_async_copy(
                    x_ref.at[0, pl.ds(off + c * grows, grows), my_col],
                    big_buf.at[i % 2],
                    big_sems.at[i % 2],
                )

            def consume(i):
                if i + 1 < len(seq):
                    add_ld(i + 1).start()
                add_ld(i).wait()
                kind, c = seq[i]
                slot = i % 2
                gr = slice(c * grows, (c + 1) * grows)
                if kind == "P":
                    if c == NG - 1:
                        rdma_x(NS - 1).wait_recv()
                    out_ref[pl.ds(p1_off + c * grows, grows), :] = (
                        big_buf[slot].astype(jnp.bfloat16) + x_recv[gr, :])
                elif c < NG - 1:
                    for s in range(c * GS, (c + 1) * GS):
                        rdma_y(s).wait_recv()
                    out_ref[pl.ds(p2_off + c * grows, grows), :] = (
                        big_buf[slot].astype(jnp.bfloat16) + y_recv[gr, :])
                else:
                    for s in range(c * GS, NS - 1):
                        rdma_y(s).wait_recv()
                        j = s - c * GS
                        out_ref[pl.ds(p2_off + s * rows, rows), :] = (
                            big_buf[slot, j * rows:(j + 1) * rows, :]
                            .astype(jnp.bfloat16)
                            + y_recv[s * rows:(s + 1) * rows, :])
                    rdma_x(NS).wait_recv()
                    out_ref[pl.ds(p2_off + (NS - 1) * rows, rows), :] = (
                        big_buf[slot, (GS - 1) * rows:GS * rows, :]
                        .astype(jnp.bfloat16)
                        + x_recv[NS * rows:(NS + 1) * rows, :])

            consume_at = {1: [0], 2: [1, 2], 3: [3, 4]}

            stage_ld(0, 0).start()
            add_ld(0).start()
            for g in range(NG):
                if g + 1 < NG:
                    stage_ld(g + 1, (g + 1) % 2).start()
                stage_ld(g, g % 2).wait()
                x_send[pl.ds(g * grows, grows), :] = (
                    ld_big[g % 2].astype(jnp.bfloat16))
                for s in range(g * GS, (g + 1) * GS):
                    rdma_x(s).start()
                if g == NG - 1:
                    ex = pltpu.make_async_copy(
                        x_ref.at[0, pl.ds(p2_off + (NS - 1) * rows, rows),
                                 their_col],
                        ld_big.at[0, 0:rows],
                        ldg_sems.at[0],
                    )
                    ex.start()
                    ex.wait()
                    x_send[NS * rows:(NS + 1) * rows, :] = (
                        ld_big[0, 0:rows, :].astype(jnp.bfloat16))
                    rdma_x(NS).start()
                if g >= 1:
                    for b in range((g - 1) * GS, g * GS):
                        rdma_x(b).wait_recv()
                        rdma_y(b).start()
                for i in consume_at.get(g, []):
                    consume(i)

            for b in range((NG - 1) * GS, NS - 1):
                rdma_x(b).wait_recv()
                rdma_y(b).start()
            for i in (5, 6, 7):
                consume(i)

        lo = slice(0, n_half)
        hi = slice(n_half, n2)

        @pl.when(my_x == 0)
        def _():
            go(lo, hi)

        @pl.when(my_x == 1)
        def _():
            go(hi, lo)

        for s in range(NS + 1):
            rdma_x(s).wait_send()
        for s in range(NS - 1):
            rdma_y(s).wait_send()

    return pl.pallas_call(
        body,
        out_shape=jax.ShapeDtypeStruct((m, n_half), jnp.bfloat16),
        in_specs=[pl.BlockSpec(memory_space=pl.ANY)],
        out_specs=pl.BlockSpec(memory_space=pltpu.VMEM),
        scratch_shapes=[
            pltpu.VMEM((2, mh // NG, n_half), jnp.float32),
            pltpu.VMEM((2, mh // NG, n_half), jnp.float32),
            pltpu.VMEM((mh + rows, n_half), jnp.bfloat16),
            pltpu.VMEM((mh + rows, n_half), jnp.bfloat16),
            pltpu.VMEM((mh, n_half), jnp.bfloat16),
            pltpu.SemaphoreType.DMA((2,)),
            pltpu.SemaphoreType.DMA((2,)),
            pltpu.SemaphoreType.DMA((NS + 1,)),
            pltpu.SemaphoreType.DMA((NS + 1,)),
            pltpu.SemaphoreType.DMA((NS,)),
            pltpu.SemaphoreType.DMA((NS,)),
        ],
        compiler_params=pltpu.CompilerParams(collective_id=0),
    )(x)


# device time: 57609 ns/iter; 1.0693x vs baseline; 1.0106x over previous
import jax
import jax.numpy as jnp
from jax import lax
from jax.experimental import pallas as pl
from jax.experimental.pallas import tpu as pltpu

NS = 16
GS = 4
NG = NS // GS


def kernel(x):
    _, m, n2 = x.shape
    n_half = n2 // 2
    mh = m // 2
    rows = mh // NS
    grows = mh // NG

    def body(x_ref, out_ref, ld_big, big_buf, x_send, x_recv, y_recv,
             ldg_sems, big_sems, xs_send, xs_recv, ys_send, ys_recv):
        my_x = lax.axis_index("x")
        my_y = lax.axis_index("y")
        my_z = lax.axis_index("z")
        x_nbr = (1 - my_x, my_y, my_z)
        y_nbr = (my_x, 1 - my_y, my_z)

        p1_off = my_y * mh
        p2_off = (1 - my_y) * mh

        barrier = pltpu.get_barrier_semaphore()
        for nbr in (x_nbr, y_nbr):
            pl.semaphore_signal(
                barrier, inc=1, device_id=nbr,
                device_id_type=pl.DeviceIdType.MESH,
            )
        pl.semaphore_wait(barrier, 2)

        def rdma_x(s):
            r = pl.ds(s * rows, rows)
            return pltpu.make_async_remote_copy(
                src_ref=x_send.at[r, :],
                dst_ref=x_recv.at[r, :],
                send_sem=xs_send.at[s],
                recv_sem=xs_recv.at[s],
                device_id=x_nbr,
                device_id_type=pl.DeviceIdType.MESH,
            )

        def rdma_y(s):
            r = pl.ds(s * rows, rows)
            return pltpu.make_async_remote_copy(
                src_ref=x_recv.at[r, :],
                dst_ref=y_recv.at[r, :],
                send_sem=ys_send.at[s],
                recv_sem=ys_recv.at[s],
                device_id=y_nbr,
                device_id_type=pl.DeviceIdType.MESH,
            )

        def go(my_col, their_col):
            def stage_ld(g, slot):
                return pltpu.make_async_copy(
                    x_ref.at[0, pl.ds(p1_off + g * grows, grows), their_col],
                    ld_big.at[slot],
                    ldg_sems.at[slot],
                )

            seq = [("P", 0), ("P", 1), ("Q", 0), ("P", 2), ("Q", 1),
                   ("P", 3), ("Q", 2), ("Q", 3)]

            def add_ld(i):
                kind, c = seq[i]
                off = p1_off if kind == "P" else p2_off
                return pltpu.make_async_copy(
                    x_ref.at[0, pl.ds(off + c * grows, grows), my_col],
                    big_buf.at[i % 2],
                    big_sems.at[i % 2],
                )

            def consume(i):
                if i + 1 < len(seq):
                    add_ld(i + 1).start()
                add_ld(i).wait()
                kind, c = seq[i]
                slot = i % 2
                gr = slice(c * grows, (c + 1) * grows)
                if kind == "P":
                    if c == NG - 1:
                        rdma_x(NS - 1).wait_recv()
                    out_ref[pl.ds(p1_off + c * grows, grows), :] = (
                        big_buf[slot].astype(jnp.bfloat16) + x_recv[gr, :])
                elif c < NG - 1:
                    for s in range(c * GS, (c + 1) * GS):
                        rdma_y(s).wait_recv()
                    out_ref[pl.ds(p2_off + c * grows, grows), :] = (
                        big_buf[slot].astype(jnp.bfloat16) + y_recv[gr, :])
                else:
                    for s in range(c * GS, NS - 1):
                        rdma_y(s).wait_recv()
                        j = s - c * GS
                        out_ref[pl.ds(p2_off + s * rows, rows), :] = (
                            big_buf[slot, j * rows:(j + 1) * rows, :]
                            .astype(jnp.bfloat16)
                            + y_recv[s * rows:(s + 1) * rows, :])
                    rdma_x(NS).wait_recv()
                    out_ref[pl.ds(p2_off + (NS - 1) * rows, rows), :] = (
                        big_buf[slot, (GS - 1) * rows:GS * rows, :]
                        .astype(jnp.bfloat16)
                        + x_recv[NS * rows:(NS + 1) * rows, :])

            consume_at = {1: [0], 2: [1, 2], 3: [3, 4]}

            head = pltpu.make_async_copy(
                x_ref.at[0, pl.ds(p1_off, rows), their_col],
                ld_big.at[0, 0:rows],
                ldg_sems.at[0],
            )
            head.start()
            rest = pltpu.make_async_copy(
                x_ref.at[0, pl.ds(p1_off + rows, grows - rows), their_col],
                ld_big.at[0, rows:grows],
                ldg_sems.at[1],
            )
            rest.start()
            add_ld(0).start()
            head.wait()
            x_send[0:rows, :] = ld_big[0, 0:rows, :].astype(jnp.bfloat16)
            rdma_x(0).start()
            rest.wait()
            x_send[rows:grows, :] = (
                ld_big[0, rows:grows, :].astype(jnp.bfloat16))
            for s in range(1, GS):
                rdma_x(s).start()
            for g in range(NG):
                if g + 1 < NG:
                    stage_ld(g + 1, (g + 1) % 2).start()
                if g >= 1:
                    stage_ld(g, g % 2).wait()
                    x_send[pl.ds(g * grows, grows), :] = (
                        ld_big[g % 2].astype(jnp.bfloat16))
                    for s in range(g * GS, (g + 1) * GS):
                        rdma_x(s).start()
                if g == NG - 1:
                    ex = pltpu.make_async_copy(
                        x_ref.at[0, pl.ds(p2_off + (NS - 1) * rows, rows),
                                 their_col],
                        ld_big.at[0, 0:rows],
                        ldg_sems.at[0],
                    )
                    ex.start()
                    ex.wait()
                    x_send[NS * rows:(NS + 1) * rows, :] = (
                        ld_big[0, 0:rows, :].astype(jnp.bfloat16))
                    rdma_x(NS).start()
                if g >= 1:
                    for b in range((g - 1) * GS, g * GS):
                        rdma_x(b).wait_recv()
                        rdma_y(b).start()
                for i in consume_at.get(g, []):
                    consume(i)

            for b in range((NG - 1) * GS, NS - 1):
                rdma_x(b).wait_recv()
                rdma_y(b).start()
            for i in (5, 6, 7):
                consume(i)

        lo = slice(0, n_half)
        hi = slice(n_half, n2)

        @pl.when(my_x == 0)
        def _():
            go(lo, hi)

        @pl.when(my_x == 1)
        def _():
            go(hi, lo)

        for s in range(NS + 1):
            rdma_x(s).wait_send()
        for s in range(NS - 1):
            rdma_y(s).wait_send()

    return pl.pallas_call(
        body,
        out_shape=jax.ShapeDtypeStruct((m, n_half), jnp.bfloat16),
        in_specs=[pl.BlockSpec(memory_space=pl.ANY)],
        out_specs=pl.BlockSpec(memory_space=pltpu.VMEM),
        scratch_shapes=[
            pltpu.VMEM((2, mh // NG, n_half), jnp.float32),
            pltpu.VMEM((2, mh // NG, n_half), jnp.float32),
            pltpu.VMEM((mh + rows, n_half), jnp.bfloat16),
            pltpu.VMEM((mh + rows, n_half), jnp.bfloat16),
            pltpu.VMEM((mh, n_half), jnp.bfloat16),
            pltpu.SemaphoreType.DMA((2,)),
            pltpu.SemaphoreType.DMA((2,)),
            pltpu.SemaphoreType.DMA((NS + 1,)),
            pltpu.SemaphoreType.DMA((NS + 1,)),
            pltpu.SemaphoreType.DMA((NS,)),
            pltpu.SemaphoreType.DMA((NS,)),
        ],
        compiler_params=pltpu.CompilerParams(collective_id=0),
    )(x)
